# baseline (speedup 1.0000x reference)
import jax
import jax.numpy as jnp
from jax.experimental import pallas as pl
from jax.experimental.pallas import tpu as pltpu

B, T, D, K = 16, 256, 512, 256
_C1 = 8
_RC = (T * B) // _C1
_TC = T // _C1


def _fused_kernel(f_ref, cb_ref, cbt_ref, wih_ref, bih_ref, whht_ref,
                  bhh_ref, wpt_ref, bp_ref, q_ref, idx_ref, loss_ref,
                  gi_a, gi_b, ctx_scr, f_scr, cw_scr, whh_scr, h_scr,
                  vq_smem):
    c = pl.program_id(0)

    @pl.when(c == 0)
    def _():
        cb = cb_ref[...]
        cw = jax.lax.dot_general(cb, wih_ref[...], (((1,), (1,)), ((), ())),
                                 preferred_element_type=jnp.float32)
        cw = cw + bih_ref[...]
        cw_scr[:, :2 * D] = (cw[:, :2 * D]
                             + bhh_ref[:, :2 * D]).astype(jnp.bfloat16)
        cw_scr[:, 2 * D:] = cw[:, 2 * D:].astype(jnp.bfloat16)
        whh_scr[...] = whht_ref[...].astype(jnp.bfloat16)
        h_scr[...] = jnp.zeros((B, D), jnp.float32)
        vq_smem[0, 0] = 0.0

    @pl.when(c < _C1)
    def _():
        cb = cb_ref[...]
        f3 = jnp.swapaxes(f_ref[...], 0, 1)
        f_scr[pl.ds(c * _TC, _TC)] = f3
        flat = f3.reshape(_RC, D)

        xc = jax.lax.dot_general(flat, cbt_ref[...],
                                 (((1,), (0,)), ((), ())),
                                 preferred_element_type=jnp.float32)
        x2 = jnp.sum(flat * flat, axis=1, keepdims=True)
        c2 = jnp.sum(cb * cb, axis=1)[None, :]
        d2 = jnp.maximum(x2 - 2.0 * xc + c2, 0.0)

        min_d = jnp.min(d2, axis=1, keepdims=True)
        iota_k = jax.lax.broadcasted_iota(jnp.int32, (_RC, K), 1)
        idx = jnp.min(jnp.where(d2 == min_d, iota_k, K), axis=1,
                      keepdims=True)
        idx_ref[...] = idx

        onehot = (iota_k == idx).astype(jnp.float32)
        q = jax.lax.dot_general(onehot, cb, (((1,), (0,)), ((), ())),
                                preferred_element_type=jnp.float32)
        q_ref[...] = jnp.swapaxes(q.reshape(_TC, B, D), 0, 1)

        diff = flat - q
        vq_smem[0, 0] += 1.25 * jnp.sum(diff * diff) / (T * B * D)

        gic = jax.lax.dot_general(onehot.astype(jnp.bfloat16), cw_scr[...],
                                  (((1,), (0,)), ((), ())),
                                  preferred_element_type=jnp.float32)
        gic = gic.astype(jnp.bfloat16).reshape(_TC, B, 3 * D)

        @pl.when(c % 2 == 0)
        def _():
            gi_a[...] = gic

        @pl.when(c % 2 == 1)
        def _():
            gi_b[...] = gic

    def run_gru(gi_ref):
        t0 = (c - 1) * _TC
        bhn = bhh_ref[:, 2 * D:]

        def step(t, h):
            g = gi_ref[t].astype(jnp.float32)
            gh = jax.lax.dot_general(h.astype(jnp.bfloat16), whh_scr[...],
                                     (((1,), (0,)), ((), ())),
                                     preferred_element_type=jnp.float32)
            rz = jax.nn.sigmoid(g[:, :2 * D] + gh[:, :2 * D])
            r = rz[:, :D]
            z = rz[:, D:]
            n = jnp.tanh(g[:, 2 * D:] + r * (gh[:, 2 * D:] + bhn))
            h_new = n + z * (h - n)
            ctx_scr[t0 + t] = h_new.astype(jnp.bfloat16)
            return h_new

        def step4(i, h):
            h = step(4 * i, h)
            h = step(4 * i + 1, h)
            h = step(4 * i + 2, h)
            h = step(4 * i + 3, h)
            return h

        h_scr[...] = jax.lax.fori_loop(0, _TC // 4, step4, h_scr[...])

    @pl.when((c > 0) & (c % 2 == 1))
    def _():
        run_gru(gi_a)

    @pl.when((c > 0) & (c % 2 == 0))
    def _():
        run_gru(gi_b)

    @pl.when(c == _C1)
    def _():
        wp = wpt_ref[...].astype(jnp.bfloat16)
        bp = bp_ref[...]
        nc = 4
        tc = T // nc
        cp1 = 0.0
        cp2 = 0.0
        for cc in range(nc):
            ctx = ctx_scr[cc * tc:(cc + 1) * tc].reshape(tc * B, D)
            proj = jax.lax.dot_general(ctx, wp, (((1,), (0,)), ((), ())),
                                       preferred_element_type=jnp.float32)
            proj3 = (proj + bp).reshape(tc, B, D)
            n1 = tc if cc < nc - 1 else tc - 1
            n2 = tc if cc < nc - 1 else tc - 2
            e1 = proj3[:n1] - f_scr[cc * tc + 1:cc * tc + 1 + n1]
            e2 = proj3[:n2] - f_scr[cc * tc + 2:cc * tc + 2 + n2]
            cp1 = cp1 + jnp.sum(e1 * e1)
            cp2 = cp2 + jnp.sum(e2 * e2)
        cp = 0.5 * (cp1 / ((T - 1) * B * D) + cp2 / ((T - 2) * B * D))
        loss_ref[...] = jnp.reshape(cp + vq_smem[0, 0], (1, 1))


@jax.jit
def kernel(features, codebook, W_ih, W_hh, b_ih, b_hh, W_proj, b_proj):
    last = _C1 - 1
    quantized, idx_tm, loss = pl.pallas_call(
        _fused_kernel,
        grid=(_C1 + 1,),
        in_specs=[
            pl.BlockSpec((B, _TC, D), lambda c: (0, jnp.minimum(c, last), 0)),
            pl.BlockSpec((K, D), lambda c: (0, 0)),
            pl.BlockSpec((D, K), lambda c: (0, 0)),
            pl.BlockSpec((3 * D, D), lambda c: (0, 0)),
            pl.BlockSpec((1, 3 * D), lambda c: (0, 0)),
            pl.BlockSpec((D, 3 * D), lambda c: (0, 0)),
            pl.BlockSpec((1, 3 * D), lambda c: (0, 0)),
            pl.BlockSpec((D, D), lambda c: (0, 0)),
            pl.BlockSpec((1, D), lambda c: (0, 0)),
        ],
        out_specs=[
            pl.BlockSpec((B, _TC, D), lambda c: (0, jnp.minimum(c, last), 0)),
            pl.BlockSpec((_RC, 1), lambda c: (jnp.minimum(c, last), 0)),
            pl.BlockSpec((1, 1), lambda c: (0, 0)),
        ],
        out_shape=[
            jax.ShapeDtypeStruct((B, T, D), jnp.float32),
            jax.ShapeDtypeStruct((T * B, 1), jnp.int32),
            jax.ShapeDtypeStruct((1, 1), jnp.float32),
        ],
        scratch_shapes=[
            pltpu.VMEM((_TC, B, 3 * D), jnp.bfloat16),
            pltpu.VMEM((_TC, B, 3 * D), jnp.bfloat16),
            pltpu.VMEM((T, B, D), jnp.bfloat16),
            pltpu.VMEM((T, B, D), jnp.float32),
            pltpu.VMEM((K, 3 * D), jnp.bfloat16),
            pltpu.VMEM((D, 3 * D), jnp.bfloat16),
            pltpu.VMEM((B, D), jnp.float32),
            pltpu.SMEM((1, 1), jnp.float32),
        ],
    )(features, codebook, jnp.swapaxes(codebook, 0, 1), W_ih,
      b_ih.reshape(1, -1), jnp.swapaxes(W_hh, 0, 1), b_hh.reshape(1, -1),
      jnp.swapaxes(W_proj, 0, 1), b_proj.reshape(1, -1))

    indices = jnp.swapaxes(idx_tm.reshape(T, B), 0, 1)
    return quantized, indices, loss[0, 0]
